# Initial kernel scaffold; baseline (speedup 1.0000x reference)
import jax
import jax.numpy as jnp
from jax.experimental import pallas as pl


def kernel(node_energy, local_or_ghost, batch, ptr, positions, cell, forces):
    raise NotImplementedError("write your pallas kernel here")


# speedup vs baseline: 3.7130x; 3.7130x over previous
import jax
import jax.numpy as jnp
from jax import lax
from jax.experimental import pallas as pl
from jax.experimental.pallas import tpu as pltpu
from jax.experimental.pallas import tpu_sc as plsc

N = 100000
G = 16
NS = 16
CHUNK = 6240
TAIL = N - NS * CHUNK
L = 16


def _sc_body(ne_hbm, lg_hbm, b_hbm, out_hbm, stage_hbm,
             ne_v, lg_v, b_v, acc_v, gath_v):
    s = lax.axis_index("s")
    base = pl.multiple_of(s * CHUNK, 8)

    pltpu.sync_copy(ne_hbm.at[pl.ds(base, CHUNK)], ne_v)
    pltpu.sync_copy(lg_hbm.at[pl.ds(base, CHUNK)], lg_v)
    pltpu.sync_copy(b_hbm.at[pl.ds(base, CHUNK)], b_v)

    acc_v[...] = jnp.zeros((L,), jnp.float32)

    def step(j, _):
        off = j * L
        v = ne_v[pl.ds(off, L)] * lg_v[pl.ds(off, L)]
        plsc.addupdate_scatter(acc_v, [b_v[pl.ds(off, L)]], v)
        return _

    lax.fori_loop(0, CHUNK // L, step, 0)

    @pl.when(s == 0)
    def _tail():
        pltpu.sync_copy(ne_hbm.at[pl.ds(NS * CHUNK, TAIL)], ne_v.at[pl.ds(0, TAIL)])
        pltpu.sync_copy(lg_hbm.at[pl.ds(NS * CHUNK, TAIL)], lg_v.at[pl.ds(0, TAIL)])
        pltpu.sync_copy(b_hbm.at[pl.ds(NS * CHUNK, TAIL)], b_v.at[pl.ds(0, TAIL)])

        def tstep(j, _):
            off = j * L
            v = ne_v[pl.ds(off, L)] * lg_v[pl.ds(off, L)]
            plsc.addupdate_scatter(acc_v, [b_v[pl.ds(off, L)]], v)
            return _

        lax.fori_loop(0, TAIL // L, tstep, 0)

    pltpu.sync_copy(acc_v, stage_hbm.at[s])
    plsc.subcore_barrier()

    @pl.when(s == 0)
    def _reduce():
        pltpu.sync_copy(stage_hbm, gath_v)
        tot = gath_v[0]
        for r in range(1, NS):
            tot = tot + gath_v[r]
        acc_v[...] = tot
        pltpu.sync_copy(acc_v, out_hbm)


_mesh = plsc.VectorSubcoreMesh(
    core_axis_name="c", subcore_axis_name="s", num_cores=1, num_subcores=NS)

_seg_sum = pl.kernel(
    _sc_body,
    out_type=(jax.ShapeDtypeStruct((G,), jnp.float32),
              jax.ShapeDtypeStruct((NS, L), jnp.float32)),
    mesh=_mesh,
    compiler_params=pltpu.CompilerParams(needs_layout_passes=False),
    scratch_types=[
        pltpu.VMEM((CHUNK,), jnp.float32),
        pltpu.VMEM((CHUNK,), jnp.float32),
        pltpu.VMEM((CHUNK,), jnp.int32),
        pltpu.VMEM((L,), jnp.float32),
        pltpu.VMEM((NS, L), jnp.float32),
    ],
)


def kernel(node_energy, local_or_ghost, batch, ptr, positions, cell, forces):
    total, _stage = _seg_sum(node_energy, local_or_ghost, batch.astype(jnp.int32))
    virials = jnp.zeros_like(cell)
    return (total, node_energy, forces, virials)


# speedup vs baseline: 4.2055x; 1.1326x over previous
import jax
import jax.numpy as jnp
from jax import lax
from jax.experimental import pallas as pl
from jax.experimental.pallas import tpu as pltpu
from jax.experimental.pallas import tpu_sc as plsc

N = 100000
G = 16
NS = 16
CHUNK = 6240
TAIL = N - NS * CHUNK
L = 16


def _sc_body(ne_hbm, lg_hbm, b_hbm, out_hbm, stage_hbm,
             ne_v, lg_v, b_v, acc_a, acc_b, tot_v, gath_v,
             sem1, sem2, sem3):
    s = lax.axis_index("s")
    base = pl.multiple_of(s * CHUNK, 8)

    c1 = pltpu.async_copy(ne_hbm.at[pl.ds(base, CHUNK)], ne_v, sem1)
    c2 = pltpu.async_copy(lg_hbm.at[pl.ds(base, CHUNK)], lg_v, sem2)
    c3 = pltpu.async_copy(b_hbm.at[pl.ds(base, CHUNK)], b_v, sem3)

    zero16 = jnp.zeros((L,), jnp.float32)
    for r in range(L):
        acc_a[r] = zero16
        acc_b[r] = zero16

    c1.wait()
    c2.wait()
    c3.wait()

    iota = lax.iota(jnp.int32, L)

    def step(j, _):
        off = j * (2 * L)
        v0 = ne_v[pl.ds(off, L)] * lg_v[pl.ds(off, L)]
        plsc.addupdate_scatter(acc_a, [iota, b_v[pl.ds(off, L)]], v0)
        v1 = ne_v[pl.ds(off + L, L)] * lg_v[pl.ds(off + L, L)]
        plsc.addupdate_scatter(acc_b, [iota, b_v[pl.ds(off + L, L)]], v1)
        return _

    lax.fori_loop(0, CHUNK // (2 * L), step, 0, unroll=4)

    @pl.when(s == 0)
    def _tail():
        t1 = pltpu.async_copy(ne_hbm.at[pl.ds(NS * CHUNK, TAIL)],
                              ne_v.at[pl.ds(0, TAIL)], sem1)
        t2 = pltpu.async_copy(lg_hbm.at[pl.ds(NS * CHUNK, TAIL)],
                              lg_v.at[pl.ds(0, TAIL)], sem2)
        t3 = pltpu.async_copy(b_hbm.at[pl.ds(NS * CHUNK, TAIL)],
                              b_v.at[pl.ds(0, TAIL)], sem3)
        t1.wait()
        t2.wait()
        t3.wait()

        def tstep(j, _):
            off = j * L
            v = ne_v[pl.ds(off, L)] * lg_v[pl.ds(off, L)]
            plsc.addupdate_scatter(acc_a, [iota, b_v[pl.ds(off, L)]], v)
            return _

        lax.fori_loop(0, TAIL // L, tstep, 0)

    tot = acc_a[0] + acc_b[0]
    for r in range(1, L):
        tot = tot + acc_a[r] + acc_b[r]
    tot_v[...] = tot

    pltpu.sync_copy(tot_v, stage_hbm.at[s])
    plsc.subcore_barrier()

    @pl.when(s == 0)
    def _reduce():
        pltpu.sync_copy(stage_hbm, gath_v)
        red = gath_v[0]
        for r in range(1, NS):
            red = red + gath_v[r]
        tot_v[...] = red
        pltpu.sync_copy(tot_v, out_hbm)


_mesh = plsc.VectorSubcoreMesh(
    core_axis_name="c", subcore_axis_name="s", num_cores=1, num_subcores=NS)

_seg_sum = pl.kernel(
    _sc_body,
    out_type=(jax.ShapeDtypeStruct((G,), jnp.float32),
              jax.ShapeDtypeStruct((NS, L), jnp.float32)),
    mesh=_mesh,
    compiler_params=pltpu.CompilerParams(needs_layout_passes=False),
    scratch_types=[
        pltpu.VMEM((CHUNK,), jnp.float32),
        pltpu.VMEM((CHUNK,), jnp.float32),
        pltpu.VMEM((CHUNK,), jnp.int32),
        pltpu.VMEM((L, L), jnp.float32),
        pltpu.VMEM((L, L), jnp.float32),
        pltpu.VMEM((L,), jnp.float32),
        pltpu.VMEM((NS, L), jnp.float32),
        pltpu.SemaphoreType.DMA,
        pltpu.SemaphoreType.DMA,
        pltpu.SemaphoreType.DMA,
    ],
)


def kernel(node_energy, local_or_ghost, batch, ptr, positions, cell, forces):
    total, _stage = _seg_sum(node_energy, local_or_ghost, batch.astype(jnp.int32))
    virials = jnp.zeros_like(cell)
    return (total, node_energy, forces, virials)


# speedup vs baseline: 4.2081x; 1.0006x over previous
import jax
import jax.numpy as jnp
from jax import lax
from jax.experimental import pallas as pl
from jax.experimental.pallas import tpu as pltpu
from jax.experimental.pallas import tpu_sc as plsc

N = 100000
G = 16
NS = 16
CHUNK = 6240
TAIL = N - NS * CHUNK
L = 16


def _sc_body(ne_hbm, lg_hbm, b_hbm, out_hbm, stage_hbm,
             ne_v, lg_v, b_v, acc_a, acc_b, acc_c, acc_d, tot_v, gath_v,
             sem1, sem2, sem3):
    s = lax.axis_index("s")
    base = pl.multiple_of(s * CHUNK, 8)

    c1 = pltpu.async_copy(ne_hbm.at[pl.ds(base, CHUNK)], ne_v, sem1)
    c2 = pltpu.async_copy(lg_hbm.at[pl.ds(base, CHUNK)], lg_v, sem2)
    c3 = pltpu.async_copy(b_hbm.at[pl.ds(base, CHUNK)], b_v, sem3)

    zero16 = jnp.zeros((L,), jnp.float32)
    for r in range(L):
        acc_a[r] = zero16
        acc_b[r] = zero16
        acc_c[r] = zero16
        acc_d[r] = zero16

    c1.wait()
    c2.wait()
    c3.wait()

    iota = lax.iota(jnp.int32, L)
    accs = (acc_a, acc_b, acc_c, acc_d)

    def step(j, _):
        off = j * (4 * L)
        for u in range(4):
            o = off + u * L
            v = ne_v[pl.ds(o, L)] * lg_v[pl.ds(o, L)]
            plsc.addupdate_scatter(accs[u], [iota, b_v[pl.ds(o, L)]], v)
        return _

    lax.fori_loop(0, CHUNK // (4 * L), step, 0, unroll=2)

    for o in range((CHUNK // (4 * L)) * 4 * L, CHUNK, L):
        v = ne_v[pl.ds(o, L)] * lg_v[pl.ds(o, L)]
        plsc.addupdate_scatter(accs[(o // L) % 4], [iota, b_v[pl.ds(o, L)]], v)

    @pl.when(s == 0)
    def _tail():
        t1 = pltpu.async_copy(ne_hbm.at[pl.ds(NS * CHUNK, TAIL)],
                              ne_v.at[pl.ds(0, TAIL)], sem1)
        t2 = pltpu.async_copy(lg_hbm.at[pl.ds(NS * CHUNK, TAIL)],
                              lg_v.at[pl.ds(0, TAIL)], sem2)
        t3 = pltpu.async_copy(b_hbm.at[pl.ds(NS * CHUNK, TAIL)],
                              b_v.at[pl.ds(0, TAIL)], sem3)
        t1.wait()
        t2.wait()
        t3.wait()

        def tstep(j, _):
            off = j * L
            v = ne_v[pl.ds(off, L)] * lg_v[pl.ds(off, L)]
            plsc.addupdate_scatter(acc_a, [iota, b_v[pl.ds(off, L)]], v)
            return _

        lax.fori_loop(0, TAIL // L, tstep, 0)

    tot = (acc_a[0] + acc_b[0]) + (acc_c[0] + acc_d[0])
    for r in range(1, L):
        tot = tot + (acc_a[r] + acc_b[r]) + (acc_c[r] + acc_d[r])
    tot_v[...] = tot

    pltpu.sync_copy(tot_v, stage_hbm.at[s])
    plsc.subcore_barrier()

    @pl.when(s == 0)
    def _reduce():
        pltpu.sync_copy(stage_hbm, gath_v)
        red = gath_v[0]
        for r in range(1, NS):
            red = red + gath_v[r]
        tot_v[...] = red
        pltpu.sync_copy(tot_v, out_hbm)


_mesh = plsc.VectorSubcoreMesh(
    core_axis_name="c", subcore_axis_name="s", num_cores=1, num_subcores=NS)

_seg_sum = pl.kernel(
    _sc_body,
    out_type=(jax.ShapeDtypeStruct((G,), jnp.float32),
              jax.ShapeDtypeStruct((NS, L), jnp.float32)),
    mesh=_mesh,
    compiler_params=pltpu.CompilerParams(needs_layout_passes=False),
    scratch_types=[
        pltpu.VMEM((CHUNK,), jnp.float32),
        pltpu.VMEM((CHUNK,), jnp.float32),
        pltpu.VMEM((CHUNK,), jnp.int32),
        pltpu.VMEM((L, L), jnp.float32),
        pltpu.VMEM((L, L), jnp.float32),
        pltpu.VMEM((L, L), jnp.float32),
        pltpu.VMEM((L, L), jnp.float32),
        pltpu.VMEM((L,), jnp.float32),
        pltpu.VMEM((NS, L), jnp.float32),
        pltpu.SemaphoreType.DMA,
        pltpu.SemaphoreType.DMA,
        pltpu.SemaphoreType.DMA,
    ],
)


def kernel(node_energy, local_or_ghost, batch, ptr, positions, cell, forces):
    total, _stage = _seg_sum(node_energy, local_or_ghost, batch.astype(jnp.int32))
    virials = jnp.zeros_like(cell)
    return (total, node_energy, forces, virials)


# speedup vs baseline: 4.2820x; 1.0176x over previous
import jax
import jax.numpy as jnp
from jax import lax
from jax.experimental import pallas as pl
from jax.experimental.pallas import tpu as pltpu
from jax.experimental.pallas import tpu_sc as plsc

N = 100000
G = 16
NC = 2
NS = 16
CHUNK = 3120
TAIL = N - NC * NS * CHUNK
L = 16
GROUPS = CHUNK // (4 * L)
REST = CHUNK - GROUPS * 4 * L


def _sc_body(ne_hbm, lg_hbm, b_hbm, out_hbm, stage_hbm,
             ne_v, lg_v, b_v, acc_a, acc_b, acc_c, acc_d, tot_v, gath_v,
             sem1, sem2, sem3):
    c = lax.axis_index("c")
    s = lax.axis_index("s")
    w = c * NS + s
    base = pl.multiple_of(w * CHUNK, 8)

    c1 = pltpu.async_copy(ne_hbm.at[pl.ds(base, CHUNK)], ne_v, sem1)
    c2 = pltpu.async_copy(lg_hbm.at[pl.ds(base, CHUNK)], lg_v, sem2)
    c3 = pltpu.async_copy(b_hbm.at[pl.ds(base, CHUNK)], b_v, sem3)

    zero16 = jnp.zeros((L,), jnp.float32)
    for r in range(L):
        acc_a[r] = zero16
        acc_b[r] = zero16
        acc_c[r] = zero16
        acc_d[r] = zero16

    c1.wait()
    c2.wait()
    c3.wait()

    iota = lax.iota(jnp.int32, L)
    accs = (acc_a, acc_b, acc_c, acc_d)

    def step(j, _):
        off = j * (4 * L)
        for u in range(4):
            o = off + u * L
            v = ne_v[pl.ds(o, L)] * lg_v[pl.ds(o, L)]
            plsc.addupdate_scatter(accs[u], [iota, b_v[pl.ds(o, L)]], v)
        return _

    lax.fori_loop(0, GROUPS, step, 0, unroll=2)

    for o in range(GROUPS * 4 * L, CHUNK, L):
        v = ne_v[pl.ds(o, L)] * lg_v[pl.ds(o, L)]
        plsc.addupdate_scatter(accs[(o // L) % 4], [iota, b_v[pl.ds(o, L)]], v)

    @pl.when(w == 0)
    def _tail():
        t1 = pltpu.async_copy(ne_hbm.at[pl.ds(NC * NS * CHUNK, TAIL)],
                              ne_v.at[pl.ds(0, TAIL)], sem1)
        t2 = pltpu.async_copy(lg_hbm.at[pl.ds(NC * NS * CHUNK, TAIL)],
                              lg_v.at[pl.ds(0, TAIL)], sem2)
        t3 = pltpu.async_copy(b_hbm.at[pl.ds(NC * NS * CHUNK, TAIL)],
                              b_v.at[pl.ds(0, TAIL)], sem3)
        t1.wait()
        t2.wait()
        t3.wait()

        def tstep(j, _):
            off = j * L
            v = ne_v[pl.ds(off, L)] * lg_v[pl.ds(off, L)]
            plsc.addupdate_scatter(acc_a, [iota, b_v[pl.ds(off, L)]], v)
            return _

        lax.fori_loop(0, TAIL // L, tstep, 0)

    tot = (acc_a[0] + acc_b[0]) + (acc_c[0] + acc_d[0])
    for r in range(1, L):
        tot = tot + (acc_a[r] + acc_b[r]) + (acc_c[r] + acc_d[r])
    tot_v[...] = tot

    pltpu.sync_copy(tot_v, stage_hbm.at[c, s])
    plsc.subcore_barrier()

    @pl.when(s == 0)
    def _reduce():
        pltpu.sync_copy(stage_hbm.at[c], gath_v)
        red = gath_v[0]
        for r in range(1, NS):
            red = red + gath_v[r]
        tot_v[...] = red
        pltpu.sync_copy(tot_v, out_hbm.at[c])


_mesh = plsc.VectorSubcoreMesh(
    core_axis_name="c", subcore_axis_name="s", num_cores=NC, num_subcores=NS)

_seg_sum = pl.kernel(
    _sc_body,
    out_type=(jax.ShapeDtypeStruct((NC, G), jnp.float32),
              jax.ShapeDtypeStruct((NC, NS, L), jnp.float32)),
    mesh=_mesh,
    compiler_params=pltpu.CompilerParams(needs_layout_passes=False),
    scratch_types=[
        pltpu.VMEM((CHUNK,), jnp.float32),
        pltpu.VMEM((CHUNK,), jnp.float32),
        pltpu.VMEM((CHUNK,), jnp.int32),
        pltpu.VMEM((L, L), jnp.float32),
        pltpu.VMEM((L, L), jnp.float32),
        pltpu.VMEM((L, L), jnp.float32),
        pltpu.VMEM((L, L), jnp.float32),
        pltpu.VMEM((L,), jnp.float32),
        pltpu.VMEM((NS, L), jnp.float32),
        pltpu.SemaphoreType.DMA,
        pltpu.SemaphoreType.DMA,
        pltpu.SemaphoreType.DMA,
    ],
)


def kernel(node_energy, local_or_ghost, batch, ptr, positions, cell, forces):
    tot2, _stage = _seg_sum(node_energy, local_or_ghost, batch.astype(jnp.int32))
    total = tot2[0] + tot2[1]
    virials = jnp.zeros_like(cell)
    return (total, node_energy, forces, virials)


# speedup vs baseline: 4.4031x; 1.0283x over previous
import jax
import jax.numpy as jnp
from jax import lax
from jax.experimental import pallas as pl
from jax.experimental.pallas import tpu as pltpu
from jax.experimental.pallas import tpu_sc as plsc

N = 100000
G = 16
NC = 2
NS = 16
CHUNK = 3120
HALF = 1568
TAIL = N - NC * NS * CHUNK
L = 16
NW = NC * NS


def _accum_range(ne_v, lg_v, b_v, accs, iota, lo, hi):
    pairs = (hi - lo) // (2 * L)

    def step(j, _):
        off = lo + j * (2 * L)
        v0 = ne_v[pl.ds(off, L)] * lg_v[pl.ds(off, L)]
        plsc.addupdate_scatter(accs[0], [iota, b_v[pl.ds(off, L)]], v0)
        v1 = ne_v[pl.ds(off + L, L)] * lg_v[pl.ds(off + L, L)]
        plsc.addupdate_scatter(accs[1], [iota, b_v[pl.ds(off + L, L)]], v1)
        return _

    lax.fori_loop(0, pairs, step, 0, unroll=4)
    for o in range(lo + pairs * 2 * L, hi, L):
        v = ne_v[pl.ds(o, L)] * lg_v[pl.ds(o, L)]
        plsc.addupdate_scatter(accs[(o // L) % 2], [iota, b_v[pl.ds(o, L)]], v)


def _sc_body(ne_hbm, lg_hbm, b_hbm, out_hbm, stage_hbm,
             ne_v, lg_v, b_v, acc_a, acc_b, tot_v, gath_v,
             sem1, sem2, sem3, sem4, sem5, sem6):
    c = lax.axis_index("c")
    s = lax.axis_index("s")
    w = c * NS + s
    base = pl.multiple_of(w * CHUNK, 8)
    base2 = pl.multiple_of(base + HALF, 8)
    REST = CHUNK - HALF

    c1 = pltpu.async_copy(ne_hbm.at[pl.ds(base, HALF)], ne_v.at[pl.ds(0, HALF)], sem1)
    c2 = pltpu.async_copy(lg_hbm.at[pl.ds(base, HALF)], lg_v.at[pl.ds(0, HALF)], sem2)
    c3 = pltpu.async_copy(b_hbm.at[pl.ds(base, HALF)], b_v.at[pl.ds(0, HALF)], sem3)
    c4 = pltpu.async_copy(ne_hbm.at[pl.ds(base2, REST)], ne_v.at[pl.ds(HALF, REST)], sem4)
    c5 = pltpu.async_copy(lg_hbm.at[pl.ds(base2, REST)], lg_v.at[pl.ds(HALF, REST)], sem5)
    c6 = pltpu.async_copy(b_hbm.at[pl.ds(base2, REST)], b_v.at[pl.ds(HALF, REST)], sem6)

    zero16 = jnp.zeros((L,), jnp.float32)
    for r in range(L):
        acc_a[r] = zero16
        acc_b[r] = zero16

    iota = lax.iota(jnp.int32, L)
    accs = (acc_a, acc_b)

    c1.wait()
    c2.wait()
    c3.wait()
    _accum_range(ne_v, lg_v, b_v, accs, iota, 0, HALF)

    c4.wait()
    c5.wait()
    c6.wait()
    _accum_range(ne_v, lg_v, b_v, accs, iota, HALF, CHUNK)

    @pl.when(w == NW - 1)
    def _tail():
        t1 = pltpu.async_copy(ne_hbm.at[pl.ds(NW * CHUNK, TAIL)],
                              ne_v.at[pl.ds(0, TAIL)], sem1)
        t2 = pltpu.async_copy(lg_hbm.at[pl.ds(NW * CHUNK, TAIL)],
                              lg_v.at[pl.ds(0, TAIL)], sem2)
        t3 = pltpu.async_copy(b_hbm.at[pl.ds(NW * CHUNK, TAIL)],
                              b_v.at[pl.ds(0, TAIL)], sem3)
        t1.wait()
        t2.wait()
        t3.wait()
        _accum_range(ne_v, lg_v, b_v, accs, iota, 0, TAIL)

    tot = acc_a[0] + acc_b[0]
    for r in range(1, L):
        tot = tot + acc_a[r] + acc_b[r]
    tot_v[...] = tot

    pltpu.sync_copy(tot_v, stage_hbm.at[c, s])
    plsc.subcore_barrier()

    @pl.when(s == 0)
    def _reduce():
        pltpu.sync_copy(stage_hbm.at[c], gath_v)
        red = gath_v[0]
        for r in range(1, NS):
            red = red + gath_v[r]
        tot_v[...] = red
        pltpu.sync_copy(tot_v, out_hbm.at[c])


_mesh = plsc.VectorSubcoreMesh(
    core_axis_name="c", subcore_axis_name="s", num_cores=NC, num_subcores=NS)

_seg_sum = pl.kernel(
    _sc_body,
    out_type=(jax.ShapeDtypeStruct((NC, G), jnp.float32),
              jax.ShapeDtypeStruct((NC, NS, L), jnp.float32)),
    mesh=_mesh,
    compiler_params=pltpu.CompilerParams(needs_layout_passes=False),
    scratch_types=[
        pltpu.VMEM((CHUNK,), jnp.float32),
        pltpu.VMEM((CHUNK,), jnp.float32),
        pltpu.VMEM((CHUNK,), jnp.int32),
        pltpu.VMEM((L, L), jnp.float32),
        pltpu.VMEM((L, L), jnp.float32),
        pltpu.VMEM((L,), jnp.float32),
        pltpu.VMEM((NS, L), jnp.float32),
        pltpu.SemaphoreType.DMA,
        pltpu.SemaphoreType.DMA,
        pltpu.SemaphoreType.DMA,
        pltpu.SemaphoreType.DMA,
        pltpu.SemaphoreType.DMA,
        pltpu.SemaphoreType.DMA,
    ],
)


def kernel(node_energy, local_or_ghost, batch, ptr, positions, cell, forces):
    tot2, _stage = _seg_sum(node_energy, local_or_ghost, batch.astype(jnp.int32))
    total = tot2[0] + tot2[1]
    virials = jnp.zeros_like(cell)
    return (total, node_energy, forces, virials)
